# baseline (speedup 1.0000x reference)
import numpy as np
import jax
import jax.numpy as jnp
from jax import lax
from jax.experimental import pallas as pl

_HW = 224.0 * 224.0
_EPS = 1e-8


def _resize_mat(out_n=224, in_n=16):
    scale = in_n / out_n
    sample = (np.arange(out_n) + 0.5) * scale - 0.5
    x = np.abs(sample[:, None] - np.arange(in_n)[None, :])
    w = np.maximum(0.0, 1.0 - x)
    return (w / w.sum(axis=1, keepdims=True)).astype(np.float32)


def _block_mat(out_n=224, in_n=16):
    w = np.zeros((out_n, in_n), np.float64)
    w[np.arange(out_n), np.arange(out_n) // (out_n // in_n)] = 1.0
    return w.astype(np.float32)


_A = _resize_mat()
_ABLK = _block_mat()
_AC = np.concatenate([_A, _ABLK], axis=1)
_AVEC = _A.sum(axis=0)
_AOUT = np.outer(_AVEC, _AVEC).reshape(1, 256).astype(np.float32)
_B2 = _A.astype(np.float64).T @ _A.astype(np.float64)
_K2 = np.kron(_B2, _B2).astype(np.float32)

_vv = np.arange(257, dtype=np.float64)
_xx = 256.0 - _vv[:, None] - _vv[None, :]
_FM = (-(_xx / 65536.0) * np.log(np.maximum(_xx, 0.0) / 65536.0 + _EPS)
       ).astype(np.float32)

_CB = 32
_CH = 16
_TH = 32


def _pass1_kernel(fp_ref, ac_ref, g_ref, s_ref, ss_ref, p_ref):
    X = fp_ref[0]
    AC = ac_ref[...]
    Y = lax.dot_general(X.reshape(_CB * 224, 224), AC,
                        (((1,), (0,)), ((), ())),
                        preferred_element_type=jnp.float32)
    Y3 = Y.reshape(_CB, 224, 32)
    Gall = lax.dot_general(Y3, AC, (((1,), (0,)), ((), ())),
                           preferred_element_type=jnp.float32)
    G1 = Gall[:, :16, :16]
    Pc = Gall[:, 16:, 16:]
    g_ref[0] = G1
    s_ref[...] = jnp.sum(Pc, axis=(1, 2)).reshape(1, 1, 1, _CB)
    ss_ref[...] = jnp.sum(X * X, axis=(1, 2)).reshape(1, 1, 1, _CB)
    psum = jnp.sum(Pc, axis=0).reshape(1, 16, 16)

    @pl.when(pl.program_id(1) == 0)
    def _():
        p_ref[...] = psum

    @pl.when(pl.program_id(1) != 0)
    def _():
        p_ref[...] += psum


def _f_ent(x):
    p = x * (1.0 / 65536.0)
    return -(p * jnp.log(p + _EPS))


def _f_safe(x):
    p = x * (1.0 / 65536.0)
    return -(p * jnp.log(jnp.maximum(x, 0.0) * (1.0 / 65536.0) + _EPS))


def _stats_kernel(fms4_ref, fm2_ref, g2_ref, p2_ref, s_ref, ss_ref,
                  fm_ref, k2_ref, aout_ref, relms_ref, wsel_ref):
    Fm = fm_ref[...]
    K2 = k2_ref[...]
    aout = aout_ref[...]

    sums_ch_cols = []
    sums_ch_rows = []
    n_uniques = []
    for b in range(2):
        fm2b = fm2_ref[b]
        mn = jnp.min(fm2b, axis=1, keepdims=True)
        mx = jnp.max(fm2b, axis=1, keepdims=True)
        msn = ((fm2b - mn) / (mx - mn) * 255.0).astype(jnp.int32)

        pvr = p2_ref[b][None, :] * (1.0 / (196.0 * 96.0))
        pmn = jnp.min(pvr)
        pmx = jnp.max(pvr)
        ppn = ((pvr - pmn) / (pmx - pmn) * 255.0).astype(jnp.int32)

        Epp3 = (ppn[:, :, None] == ppn[:, None, :])
        m_k = jnp.sum(Epp3.astype(jnp.float32), axis=2)
        iota_b = lax.broadcasted_iota(jnp.int32, (1, 256, 256), 2)
        mh = jnp.sum((ppn[:, :, None] == iota_b).astype(jnp.float32),
                     axis=1)
        iota_v1 = lax.broadcasted_iota(jnp.int32, (1, 256, 257), 2)
        cm = jnp.sum((mh.astype(jnp.int32)[:, :, None] == iota_v1
                      ).astype(jnp.float32), axis=1)
        u = lax.dot_general(cm, Fm, (((1,), (0,)), ((), ())),
                            preferred_element_type=jnp.float32)

        mi_parts = []
        for c0 in range(0, 96, _CH):
            msn_c = msn[c0:c0 + _CH]
            eq3 = (msn_c[:, :, None]
                   == lax.broadcasted_iota(jnp.int32, (_CH, 256, 256), 2))
            n_c = jnp.sum(eq3.astype(jnp.float32), axis=1)
            pn = n_c * (1.0 / 256.0)
            h_ms_c = -jnp.sum(pn * jnp.log(pn + _EPS), axis=1,
                              keepdims=True)
            eqv = (n_c.astype(jnp.int32)[:, :, None]
                   == lax.broadcasted_iota(jnp.int32, (_CH, 256, 257), 2))
            cn_c = jnp.sum(eqv.astype(jnp.float32), axis=1)
            base_c = lax.dot_general(cn_c, u, (((1,), (1,)), ((), ())),
                                     preferred_element_type=jnp.float32)
            Ems = (msn_c[:, :, None] == msn_c[:, None, :])
            n_k = jnp.sum(Ems.astype(jnp.float32), axis=2)
            Jk = jnp.sum((Ems & Epp3).astype(jnp.float32), axis=2)
            t1 = 256.0 - n_k - m_k
            corr_c = jnp.sum((_f_ent(t1 + 2.0 * Jk) - _f_safe(t1)) / Jk,
                             axis=1, keepdims=True)
            mi_parts.append(h_ms_c - base_c - corr_c)
        mi = jnp.concatenate(mi_parts, axis=0)
        e = jnp.exp(mi - jnp.max(mi))
        mis = e / jnp.sum(e)
        relms_ref[b] = fms4_ref[b] * (1.0 + mis.reshape(96, 1, 1))

        mu_ms = jnp.sum(fm2b * aout, axis=1, keepdims=True) * (1.0 / _HW)
        Q = lax.dot_general(fm2b, K2, (((1,), (0,)), ((), ())),
                            preferred_element_type=jnp.float32)
        E2 = jnp.sum(Q * fm2b, axis=1, keepdims=True)
        f_ms_s = jnp.sqrt(E2 - _HW * mu_ms * mu_ms)
        s_row = s_ref[b][None, :]
        ss_row = ss_ref[b][None, :]
        mu_p = s_row * (1.0 / _HW)
        f_p_s = jnp.sqrt(ss_row - s_row * s_row * (1.0 / _HW))
        raw = lax.dot_general(fm2b, g2_ref[b], (((1,), (1,)), ((), ())),
                              preferred_element_type=jnp.float32)
        num = raw - mu_ms * (_HW * mu_p)
        denom = f_ms_s * f_p_s * 0.01
        s_m = num / denom

        idx = jnp.argmax(s_m, axis=1)
        mv = jnp.max(s_m, axis=1, keepdims=True)
        ev = jnp.exp(mv - jnp.max(mv))
        mvs = ev / jnp.sum(ev)
        iota0 = lax.broadcasted_iota(jnp.int32, (96, 96), 0)
        hit = (iota0 == idx[None, :]).astype(jnp.float32)
        s_ch_col = lax.dot_general(hit, mvs, (((1,), (0,)), ((), ())),
                                   preferred_element_type=jnp.float32)
        s_ch_row = lax.dot_general(mvs, hit, (((0,), (1,)), ((), ())),
                                   preferred_element_type=jnp.float32)
        present = (jnp.sum(hit, axis=1, keepdims=True) > 0.0)
        n_uniques.append(jnp.sum(present.astype(jnp.int32)))
        sums_ch_cols.append(s_ch_col)
        sums_ch_rows.append(s_ch_row)

    kk = (jnp.minimum(n_uniques[0], n_uniques[1]) + 1) // 2

    iota0 = lax.broadcasted_iota(jnp.int32, (96, 96), 0)
    iota1 = lax.broadcasted_iota(jnp.int32, (96, 96), 1)
    for b in range(2):
        s_col = sums_ch_cols[b]
        s_row = sums_ch_rows[b]
        gt = s_row > s_col
        eqhi = (s_row == s_col) & (iota1 > iota0)
        rank = jnp.sum((gt | eqhi).astype(jnp.int32), axis=1, keepdims=True)
        sel = rank < kk
        smax = jnp.max(s_col)
        w_e = jnp.where(sel, jnp.exp(s_col - smax), 0.0)
        wsel_ref[b] = w_e / jnp.sum(w_e)


def _pass2_kernel(fp_ref, w_ref, out_ref):
    X = fp_ref[0]
    wv = w_ref[0]
    sig = 1.0 / (1.0 + jnp.exp(-X))
    mask = jnp.sum(sig * wv, axis=0, keepdims=True)
    out_ref[0] = X * (1.0 + mask)


def kernel(f_p, f_ms):
    B, C, H, W = f_p.shape
    f32 = jnp.float32

    ac = jnp.asarray(_AC)
    G, sums, sumsq, P = pl.pallas_call(
        _pass1_kernel,
        grid=(B, C // _CB),
        in_specs=[
            pl.BlockSpec((1, _CB, H, W), lambda b, cb: (b, cb, 0, 0)),
            pl.BlockSpec((224, 32), lambda b, cb: (0, 0)),
        ],
        out_specs=[
            pl.BlockSpec((1, _CB, 16, 16), lambda b, cb: (b, cb, 0, 0)),
            pl.BlockSpec((1, 1, 1, _CB), lambda b, cb: (b, cb, 0, 0)),
            pl.BlockSpec((1, 1, 1, _CB), lambda b, cb: (b, cb, 0, 0)),
            pl.BlockSpec((1, 16, 16), lambda b, cb: (b, 0, 0)),
        ],
        out_shape=[
            jax.ShapeDtypeStruct((B, C, 16, 16), f32),
            jax.ShapeDtypeStruct((B, C // _CB, 1, _CB), f32),
            jax.ShapeDtypeStruct((B, C // _CB, 1, _CB), f32),
            jax.ShapeDtypeStruct((B, 16, 16), f32),
        ],
    )(f_p, ac)
    sums = sums.reshape(B, C)
    sumsq = sumsq.reshape(B, C)

    fm2 = jnp.swapaxes(f_ms, 2, 3).reshape(B, C, 256)
    G2 = G.reshape(B, C, 256)
    P2 = P.reshape(B, 256)

    rel_ms, wsel = pl.pallas_call(
        _stats_kernel,
        out_shape=[
            jax.ShapeDtypeStruct((B, C, 16, 16), f32),
            jax.ShapeDtypeStruct((B, C, 1), f32),
        ],
    )(f_ms, fm2, G2, P2, sums, sumsq,
      jnp.asarray(_FM), jnp.asarray(_K2), jnp.asarray(_AOUT))

    rel_p = pl.pallas_call(
        _pass2_kernel,
        grid=(B, H // _TH),
        in_specs=[
            pl.BlockSpec((1, C, _TH, W), lambda b, t: (b, 0, t, 0)),
            pl.BlockSpec((1, C, 1, 1), lambda b, t: (b, 0, 0, 0)),
        ],
        out_specs=pl.BlockSpec((1, C, _TH, W), lambda b, t: (b, 0, t, 0)),
        out_shape=jax.ShapeDtypeStruct((B, C, H, W), f32),
    )(f_p, wsel.reshape(B, C, 1, 1))

    return rel_p, rel_ms


# speedup vs baseline: 1.0013x; 1.0013x over previous
import numpy as np
import jax
import jax.numpy as jnp
from jax import lax
from jax.experimental import pallas as pl

_HW = 224.0 * 224.0
_EPS = 1e-8


def _resize_mat(out_n=224, in_n=16):
    scale = in_n / out_n
    sample = (np.arange(out_n) + 0.5) * scale - 0.5
    x = np.abs(sample[:, None] - np.arange(in_n)[None, :])
    w = np.maximum(0.0, 1.0 - x)
    return (w / w.sum(axis=1, keepdims=True)).astype(np.float32)


def _block_mat(out_n=224, in_n=16):
    w = np.zeros((out_n, in_n), np.float64)
    w[np.arange(out_n), np.arange(out_n) // (out_n // in_n)] = 1.0
    return w.astype(np.float32)


_A = _resize_mat()
_ABLK = _block_mat()
_AC = np.concatenate([_A, _ABLK], axis=1)
_AVEC = _A.sum(axis=0)
_AOUT = np.outer(_AVEC, _AVEC).reshape(1, 256).astype(np.float32)
_B2 = _A.astype(np.float64).T @ _A.astype(np.float64)
_K2 = np.kron(_B2, _B2).astype(np.float32)

_vv = np.arange(257, dtype=np.float64)
_xx = 256.0 - _vv[:, None] - _vv[None, :]
_FM = (-(_xx / 65536.0) * np.log(np.maximum(_xx, 0.0) / 65536.0 + _EPS)
       ).astype(np.float32)

_CB = 32
_CH = 16
_TH = 112


def _pass1_kernel(fp_ref, ac_ref, g_ref, s_ref, ss_ref, p_ref):
    X = fp_ref[0]
    AC = ac_ref[...]
    Y3 = lax.dot_general(X, AC, (((2,), (0,)), ((), ())),
                         preferred_element_type=jnp.float32)
    Gall = lax.dot_general(Y3, AC, (((1,), (0,)), ((), ())),
                           preferred_element_type=jnp.float32)
    G1 = Gall[:, :16, :16]
    Pc = Gall[:, 16:, 16:]
    g_ref[0] = G1
    s_ref[...] = jnp.sum(Pc, axis=(1, 2)).reshape(1, 1, 1, _CB)
    ss_ref[...] = jnp.sum(X * X, axis=(1, 2)).reshape(1, 1, 1, _CB)
    psum = jnp.sum(Pc, axis=0).reshape(1, 16, 16)

    @pl.when(pl.program_id(1) == 0)
    def _():
        p_ref[...] = psum

    @pl.when(pl.program_id(1) != 0)
    def _():
        p_ref[...] += psum


def _f_ent(x):
    p = x * (1.0 / 65536.0)
    return -(p * jnp.log(p + _EPS))


def _f_safe(x):
    p = x * (1.0 / 65536.0)
    return -(p * jnp.log(jnp.maximum(x, 0.0) * (1.0 / 65536.0) + _EPS))


def _stats_kernel(fms4_ref, fm2_ref, g2_ref, p2_ref, s_ref, ss_ref,
                  fm_ref, k2_ref, aout_ref, relms_ref, wsel_ref):
    Fm = fm_ref[...]
    K2 = k2_ref[...]
    aout = aout_ref[...]

    sums_ch_cols = []
    sums_ch_rows = []
    n_uniques = []
    for b in range(2):
        fm2b = fm2_ref[b]
        mn = jnp.min(fm2b, axis=1, keepdims=True)
        mx = jnp.max(fm2b, axis=1, keepdims=True)
        msn = ((fm2b - mn) / (mx - mn) * 255.0).astype(jnp.int32)

        pvr = p2_ref[b][None, :] * (1.0 / (196.0 * 96.0))
        pmn = jnp.min(pvr)
        pmx = jnp.max(pvr)
        ppn = ((pvr - pmn) / (pmx - pmn) * 255.0).astype(jnp.int32)

        Epp3 = (ppn[:, :, None] == ppn[:, None, :])
        m_k = jnp.sum(Epp3.astype(jnp.float32), axis=2)
        iota_b = lax.broadcasted_iota(jnp.int32, (1, 256, 256), 2)
        mh = jnp.sum((ppn[:, :, None] == iota_b).astype(jnp.float32),
                     axis=1)
        iota_v1 = lax.broadcasted_iota(jnp.int32, (1, 256, 257), 2)
        cm = jnp.sum((mh.astype(jnp.int32)[:, :, None] == iota_v1
                      ).astype(jnp.float32), axis=1)
        u = lax.dot_general(cm, Fm, (((1,), (0,)), ((), ())),
                            preferred_element_type=jnp.float32)

        mi_parts = []
        for c0 in range(0, 96, _CH):
            msn_c = msn[c0:c0 + _CH]
            eq3 = (msn_c[:, :, None]
                   == lax.broadcasted_iota(jnp.int32, (_CH, 256, 256), 2))
            n_c = jnp.sum(eq3.astype(jnp.float32), axis=1)
            pn = n_c * (1.0 / 256.0)
            h_ms_c = -jnp.sum(pn * jnp.log(pn + _EPS), axis=1,
                              keepdims=True)
            eqv = (n_c.astype(jnp.int32)[:, :, None]
                   == lax.broadcasted_iota(jnp.int32, (_CH, 256, 257), 2))
            cn_c = jnp.sum(eqv.astype(jnp.float32), axis=1)
            base_c = lax.dot_general(cn_c, u, (((1,), (1,)), ((), ())),
                                     preferred_element_type=jnp.float32)
            Ems = (msn_c[:, :, None] == msn_c[:, None, :])
            n_k = jnp.sum(Ems.astype(jnp.float32), axis=2)
            Jk = jnp.sum((Ems & Epp3).astype(jnp.float32), axis=2)
            t1 = 256.0 - n_k - m_k
            corr_c = jnp.sum((_f_ent(t1 + 2.0 * Jk) - _f_safe(t1)) / Jk,
                             axis=1, keepdims=True)
            mi_parts.append(h_ms_c - base_c - corr_c)
        mi = jnp.concatenate(mi_parts, axis=0)
        e = jnp.exp(mi - jnp.max(mi))
        mis = e / jnp.sum(e)
        relms_ref[b] = fms4_ref[b] * (1.0 + mis.reshape(96, 1, 1))

        mu_ms = jnp.sum(fm2b * aout, axis=1, keepdims=True) * (1.0 / _HW)
        Q = lax.dot_general(fm2b, K2, (((1,), (0,)), ((), ())),
                            preferred_element_type=jnp.float32)
        E2 = jnp.sum(Q * fm2b, axis=1, keepdims=True)
        f_ms_s = jnp.sqrt(E2 - _HW * mu_ms * mu_ms)
        s_row = s_ref[b][None, :]
        ss_row = ss_ref[b][None, :]
        mu_p = s_row * (1.0 / _HW)
        f_p_s = jnp.sqrt(ss_row - s_row * s_row * (1.0 / _HW))
        raw = lax.dot_general(fm2b, g2_ref[b], (((1,), (1,)), ((), ())),
                              preferred_element_type=jnp.float32)
        num = raw - mu_ms * (_HW * mu_p)
        denom = f_ms_s * f_p_s * 0.01
        s_m = num / denom

        idx = jnp.argmax(s_m, axis=1)
        mv = jnp.max(s_m, axis=1, keepdims=True)
        ev = jnp.exp(mv - jnp.max(mv))
        mvs = ev / jnp.sum(ev)
        iota0 = lax.broadcasted_iota(jnp.int32, (96, 96), 0)
        hit = (iota0 == idx[None, :]).astype(jnp.float32)
        s_ch_col = lax.dot_general(hit, mvs, (((1,), (0,)), ((), ())),
                                   preferred_element_type=jnp.float32)
        s_ch_row = lax.dot_general(mvs, hit, (((0,), (1,)), ((), ())),
                                   preferred_element_type=jnp.float32)
        present = (jnp.sum(hit, axis=1, keepdims=True) > 0.0)
        n_uniques.append(jnp.sum(present.astype(jnp.int32)))
        sums_ch_cols.append(s_ch_col)
        sums_ch_rows.append(s_ch_row)

    kk = (jnp.minimum(n_uniques[0], n_uniques[1]) + 1) // 2

    iota0 = lax.broadcasted_iota(jnp.int32, (96, 96), 0)
    iota1 = lax.broadcasted_iota(jnp.int32, (96, 96), 1)
    for b in range(2):
        s_col = sums_ch_cols[b]
        s_row = sums_ch_rows[b]
        gt = s_row > s_col
        eqhi = (s_row == s_col) & (iota1 > iota0)
        rank = jnp.sum((gt | eqhi).astype(jnp.int32), axis=1, keepdims=True)
        sel = rank < kk
        smax = jnp.max(s_col)
        w_e = jnp.where(sel, jnp.exp(s_col - smax), 0.0)
        wsel_ref[b] = w_e / jnp.sum(w_e)


def _pass2_kernel(fp_ref, w_ref, out_ref):
    X = fp_ref[0]
    wv = w_ref[0]
    sig = 1.0 / (1.0 + jnp.exp(-X))
    mask = jnp.sum(sig * wv, axis=0, keepdims=True)
    out_ref[0] = X * (1.0 + mask)


def kernel(f_p, f_ms):
    B, C, H, W = f_p.shape
    f32 = jnp.float32

    ac = jnp.asarray(_AC)
    G, sums, sumsq, P = pl.pallas_call(
        _pass1_kernel,
        grid=(B, C // _CB),
        in_specs=[
            pl.BlockSpec((1, _CB, H, W), lambda b, cb: (b, cb, 0, 0)),
            pl.BlockSpec((224, 32), lambda b, cb: (0, 0)),
        ],
        out_specs=[
            pl.BlockSpec((1, _CB, 16, 16), lambda b, cb: (b, cb, 0, 0)),
            pl.BlockSpec((1, 1, 1, _CB), lambda b, cb: (b, cb, 0, 0)),
            pl.BlockSpec((1, 1, 1, _CB), lambda b, cb: (b, cb, 0, 0)),
            pl.BlockSpec((1, 16, 16), lambda b, cb: (b, 0, 0)),
        ],
        out_shape=[
            jax.ShapeDtypeStruct((B, C, 16, 16), f32),
            jax.ShapeDtypeStruct((B, C // _CB, 1, _CB), f32),
            jax.ShapeDtypeStruct((B, C // _CB, 1, _CB), f32),
            jax.ShapeDtypeStruct((B, 16, 16), f32),
        ],
    )(f_p, ac)
    sums = sums.reshape(B, C)
    sumsq = sumsq.reshape(B, C)

    fm2 = jnp.swapaxes(f_ms, 2, 3).reshape(B, C, 256)
    G2 = G.reshape(B, C, 256)
    P2 = P.reshape(B, 256)

    rel_ms, wsel = pl.pallas_call(
        _stats_kernel,
        out_shape=[
            jax.ShapeDtypeStruct((B, C, 16, 16), f32),
            jax.ShapeDtypeStruct((B, C, 1), f32),
        ],
    )(f_ms, fm2, G2, P2, sums, sumsq,
      jnp.asarray(_FM), jnp.asarray(_K2), jnp.asarray(_AOUT))

    rel_p = pl.pallas_call(
        _pass2_kernel,
        grid=(B, H // _TH),
        in_specs=[
            pl.BlockSpec((1, C, _TH, W), lambda b, t: (b, 0, t, 0)),
            pl.BlockSpec((1, C, 1, 1), lambda b, t: (b, 0, 0, 0)),
        ],
        out_specs=pl.BlockSpec((1, C, _TH, W), lambda b, t: (b, 0, t, 0)),
        out_shape=jax.ShapeDtypeStruct((B, C, H, W), f32),
    )(f_p, wsel.reshape(B, C, 1, 1))

    return rel_p, rel_ms


# speedup vs baseline: 1.0097x; 1.0084x over previous
import numpy as np
import jax
import jax.numpy as jnp
from jax import lax
from jax.experimental import pallas as pl

_HW = 224.0 * 224.0
_EPS = 1e-8


def _resize_mat(out_n=224, in_n=16):
    scale = in_n / out_n
    sample = (np.arange(out_n) + 0.5) * scale - 0.5
    x = np.abs(sample[:, None] - np.arange(in_n)[None, :])
    w = np.maximum(0.0, 1.0 - x)
    return (w / w.sum(axis=1, keepdims=True)).astype(np.float32)


def _block_mat(out_n=224, in_n=16):
    w = np.zeros((out_n, in_n), np.float64)
    w[np.arange(out_n), np.arange(out_n) // (out_n // in_n)] = 1.0
    return w.astype(np.float32)


_A = _resize_mat()
_ABLK = _block_mat()
_AC = np.concatenate([_A, _ABLK], axis=1)
_AVEC = _A.sum(axis=0)
_AOUT = np.outer(_AVEC, _AVEC).reshape(1, 256).astype(np.float32)
_B2 = _A.astype(np.float64).T @ _A.astype(np.float64)
_K2 = np.kron(_B2, _B2).astype(np.float32)

_vv = np.arange(257, dtype=np.float64)
_xx = 256.0 - _vv[:, None] - _vv[None, :]
_FM = (-(_xx / 65536.0) * np.log(np.maximum(_xx, 0.0) / 65536.0 + _EPS)
       ).astype(np.float32)

_CB = 32
_CH = 32
_TH = 112


def _pass1_kernel(fp_ref, ac_ref, g_ref, s_ref, ss_ref, p_ref):
    X = fp_ref[0]
    AC = ac_ref[...]
    Y3 = lax.dot_general(X, AC, (((2,), (0,)), ((), ())),
                         preferred_element_type=jnp.float32)
    Gall = lax.dot_general(Y3, AC, (((1,), (0,)), ((), ())),
                           preferred_element_type=jnp.float32)
    G1 = Gall[:, :16, :16]
    Pc = Gall[:, 16:, 16:]
    g_ref[0] = G1
    s_ref[...] = jnp.sum(Pc, axis=(1, 2)).reshape(1, 1, 1, _CB)
    ss_ref[...] = jnp.sum(X * X, axis=(1, 2)).reshape(1, 1, 1, _CB)
    psum = jnp.sum(Pc, axis=0).reshape(1, 16, 16)

    @pl.when(pl.program_id(1) == 0)
    def _():
        p_ref[...] = psum

    @pl.when(pl.program_id(1) != 0)
    def _():
        p_ref[...] += psum


def _f_ent(x):
    p = x * (1.0 / 65536.0)
    return -(p * jnp.log(p + _EPS))


def _f_safe(x):
    p = x * (1.0 / 65536.0)
    return -(p * jnp.log(jnp.maximum(x, 0.0) * (1.0 / 65536.0) + _EPS))


def _stats_kernel(fms4_ref, fm2_ref, g2_ref, p2_ref, s_ref, ss_ref,
                  fm_ref, k2_ref, aout_ref, relms_ref, wsel_ref):
    Fm = fm_ref[...]
    K2 = k2_ref[...]
    aout = aout_ref[...]

    sums_ch_cols = []
    sums_ch_rows = []
    n_uniques = []
    for b in range(2):
        fm2b = fm2_ref[b]
        mn = jnp.min(fm2b, axis=1, keepdims=True)
        mx = jnp.max(fm2b, axis=1, keepdims=True)
        msn = ((fm2b - mn) / (mx - mn) * 255.0).astype(jnp.int32)

        pvr = p2_ref[b][None, :] * (1.0 / (196.0 * 96.0))
        pmn = jnp.min(pvr)
        pmx = jnp.max(pvr)
        ppn = ((pvr - pmn) / (pmx - pmn) * 255.0).astype(jnp.int32)

        Epp3 = (ppn[:, :, None] == ppn[:, None, :])
        m_k = jnp.sum(Epp3.astype(jnp.float32), axis=2)
        iota_b = lax.broadcasted_iota(jnp.int32, (1, 256, 256), 2)
        mh = jnp.sum((ppn[:, :, None] == iota_b).astype(jnp.float32),
                     axis=1)
        iota_v1 = lax.broadcasted_iota(jnp.int32, (1, 256, 257), 2)
        cm = jnp.sum((mh.astype(jnp.int32)[:, :, None] == iota_v1
                      ).astype(jnp.float32), axis=1)
        u = lax.dot_general(cm, Fm, (((1,), (0,)), ((), ())),
                            preferred_element_type=jnp.float32)

        mi_parts = []
        for c0 in range(0, 96, _CH):
            msn_c = msn[c0:c0 + _CH]
            eq3 = (msn_c[:, :, None]
                   == lax.broadcasted_iota(jnp.int32, (_CH, 256, 256), 2))
            n_c = jnp.sum(eq3.astype(jnp.float32), axis=1)
            pn = n_c * (1.0 / 256.0)
            h_ms_c = -jnp.sum(pn * jnp.log(pn + _EPS), axis=1,
                              keepdims=True)
            eqv = (n_c.astype(jnp.int32)[:, :, None]
                   == lax.broadcasted_iota(jnp.int32, (_CH, 256, 257), 2))
            cn_c = jnp.sum(eqv.astype(jnp.float32), axis=1)
            base_c = lax.dot_general(cn_c, u, (((1,), (1,)), ((), ())),
                                     preferred_element_type=jnp.float32)
            Ems = (msn_c[:, :, None] == msn_c[:, None, :])
            n_k = jnp.sum(Ems.astype(jnp.float32), axis=2)
            Jk = jnp.sum((Ems & Epp3).astype(jnp.float32), axis=2)
            t1 = 256.0 - n_k - m_k
            corr_c = jnp.sum((_f_ent(t1 + 2.0 * Jk) - _f_safe(t1)) / Jk,
                             axis=1, keepdims=True)
            mi_parts.append(h_ms_c - base_c - corr_c)
        mi = jnp.concatenate(mi_parts, axis=0)
        e = jnp.exp(mi - jnp.max(mi))
        mis = e / jnp.sum(e)
        relms_ref[b] = fms4_ref[b] * (1.0 + mis.reshape(96, 1, 1))

        mu_ms = jnp.sum(fm2b * aout, axis=1, keepdims=True) * (1.0 / _HW)
        Q = lax.dot_general(fm2b, K2, (((1,), (0,)), ((), ())),
                            preferred_element_type=jnp.float32)
        E2 = jnp.sum(Q * fm2b, axis=1, keepdims=True)
        f_ms_s = jnp.sqrt(E2 - _HW * mu_ms * mu_ms)
        s_row = s_ref[b][None, :]
        ss_row = ss_ref[b][None, :]
        mu_p = s_row * (1.0 / _HW)
        f_p_s = jnp.sqrt(ss_row - s_row * s_row * (1.0 / _HW))
        raw = lax.dot_general(fm2b, g2_ref[b], (((1,), (1,)), ((), ())),
                              preferred_element_type=jnp.float32)
        num = raw - mu_ms * (_HW * mu_p)
        denom = f_ms_s * f_p_s * 0.01
        s_m = num / denom

        idx = jnp.argmax(s_m, axis=1)
        mv = jnp.max(s_m, axis=1, keepdims=True)
        ev = jnp.exp(mv - jnp.max(mv))
        mvs = ev / jnp.sum(ev)
        iota0 = lax.broadcasted_iota(jnp.int32, (96, 96), 0)
        hit = (iota0 == idx[None, :]).astype(jnp.float32)
        s_ch_col = lax.dot_general(hit, mvs, (((1,), (0,)), ((), ())),
                                   preferred_element_type=jnp.float32)
        s_ch_row = lax.dot_general(mvs, hit, (((0,), (1,)), ((), ())),
                                   preferred_element_type=jnp.float32)
        present = (jnp.sum(hit, axis=1, keepdims=True) > 0.0)
        n_uniques.append(jnp.sum(present.astype(jnp.int32)))
        sums_ch_cols.append(s_ch_col)
        sums_ch_rows.append(s_ch_row)

    kk = (jnp.minimum(n_uniques[0], n_uniques[1]) + 1) // 2

    iota0 = lax.broadcasted_iota(jnp.int32, (96, 96), 0)
    iota1 = lax.broadcasted_iota(jnp.int32, (96, 96), 1)
    for b in range(2):
        s_col = sums_ch_cols[b]
        s_row = sums_ch_rows[b]
        gt = s_row > s_col
        eqhi = (s_row == s_col) & (iota1 > iota0)
        rank = jnp.sum((gt | eqhi).astype(jnp.int32), axis=1, keepdims=True)
        sel = rank < kk
        smax = jnp.max(s_col)
        w_e = jnp.where(sel, jnp.exp(s_col - smax), 0.0)
        wsel_ref[b] = w_e / jnp.sum(w_e)


def _pass2_kernel(fp_ref, w_ref, out_ref):
    X = fp_ref[0]
    wv = w_ref[0]
    sig = 1.0 / (1.0 + jnp.exp(-X))
    mask = jnp.sum(sig * wv, axis=0, keepdims=True)
    out_ref[0] = X * (1.0 + mask)


def kernel(f_p, f_ms):
    B, C, H, W = f_p.shape
    f32 = jnp.float32

    ac = jnp.asarray(_AC)
    G, sums, sumsq, P = pl.pallas_call(
        _pass1_kernel,
        grid=(B, C // _CB),
        in_specs=[
            pl.BlockSpec((1, _CB, H, W), lambda b, cb: (b, cb, 0, 0)),
            pl.BlockSpec((224, 32), lambda b, cb: (0, 0)),
        ],
        out_specs=[
            pl.BlockSpec((1, _CB, 16, 16), lambda b, cb: (b, cb, 0, 0)),
            pl.BlockSpec((1, 1, 1, _CB), lambda b, cb: (b, cb, 0, 0)),
            pl.BlockSpec((1, 1, 1, _CB), lambda b, cb: (b, cb, 0, 0)),
            pl.BlockSpec((1, 16, 16), lambda b, cb: (b, 0, 0)),
        ],
        out_shape=[
            jax.ShapeDtypeStruct((B, C, 16, 16), f32),
            jax.ShapeDtypeStruct((B, C // _CB, 1, _CB), f32),
            jax.ShapeDtypeStruct((B, C // _CB, 1, _CB), f32),
            jax.ShapeDtypeStruct((B, 16, 16), f32),
        ],
    )(f_p, ac)
    sums = sums.reshape(B, C)
    sumsq = sumsq.reshape(B, C)

    fm2 = jnp.swapaxes(f_ms, 2, 3).reshape(B, C, 256)
    G2 = G.reshape(B, C, 256)
    P2 = P.reshape(B, 256)

    rel_ms, wsel = pl.pallas_call(
        _stats_kernel,
        out_shape=[
            jax.ShapeDtypeStruct((B, C, 16, 16), f32),
            jax.ShapeDtypeStruct((B, C, 1), f32),
        ],
    )(f_ms, fm2, G2, P2, sums, sumsq,
      jnp.asarray(_FM), jnp.asarray(_K2), jnp.asarray(_AOUT))

    rel_p = pl.pallas_call(
        _pass2_kernel,
        grid=(B, H // _TH),
        in_specs=[
            pl.BlockSpec((1, C, _TH, W), lambda b, t: (b, 0, t, 0)),
            pl.BlockSpec((1, C, 1, 1), lambda b, t: (b, 0, 0, 0)),
        ],
        out_specs=pl.BlockSpec((1, C, _TH, W), lambda b, t: (b, 0, t, 0)),
        out_shape=jax.ShapeDtypeStruct((B, C, H, W), f32),
    )(f_p, wsel.reshape(B, C, 1, 1))

    return rel_p, rel_ms
